# Initial kernel scaffold; baseline (speedup 1.0000x reference)
import jax
import jax.numpy as jnp
from jax.experimental import pallas as pl


def kernel(input_ids, table):
    raise NotImplementedError("write your pallas kernel here")


# speedup vs baseline: 14.4003x; 14.4003x over previous
import jax
import jax.numpy as jnp
from jax import lax
from jax.experimental import pallas as pl
from jax.experimental.pallas import tpu as pltpu
from jax.experimental.pallas import tpu_sc as plsc

B = 4096
S = 200
D = 128
L = 16
NC = 2
NS = 16
NW = NC * NS
BPW = B // NW
ND = D // L
S0 = 128
S1 = S - S0
_NFULL = S // L
_TAIL = S - L
_TAIL_NEW = _NFULL * L - _TAIL


def _encoder_body(ids_hbm, table_hbm, out_hbm, idx_v, rows_v, out_v, sem0, sem1):
    wid = lax.axis_index("s") * NC + lax.axis_index("c")
    base = wid * BPW
    pltpu.sync_copy(ids_hbm.at[pl.ds(base, BPW), :], idx_v)

    sems = (sem0, sem1)

    def start_gather(b, k):
        pltpu.async_copy(table_hbm.at[idx_v.at[b, pl.ds(0, S0)]],
                         rows_v.at[k, pl.ds(0, S0)], sems[k])
        pltpu.async_copy(table_hbm.at[idx_v.at[b, pl.ds(S0, S1)]],
                         rows_v.at[k, pl.ds(S0, S1)], sems[k])

    def wait_gather(b, k):
        pltpu.make_async_copy(table_hbm.at[idx_v.at[b, pl.ds(0, S0)]],
                              rows_v.at[k, pl.ds(0, S0)], sems[k]).wait()
        pltpu.make_async_copy(table_hbm.at[idx_v.at[b, pl.ds(S0, S1)]],
                              rows_v.at[k, pl.ds(S0, S1)], sems[k]).wait()

    def seq_len_splat(b):
        cnt = jnp.zeros((L,), jnp.int32)
        for c in range(_NFULL):
            v = idx_v[b, pl.ds(c * L, L)]
            cnt = cnt + plsc.all_reduce_population_count(v != 0)
        v = idx_v[b, pl.ds(_TAIL, L)]
        fresh = (v != 0) & (lax.iota(jnp.int32, L) >= _TAIL_NEW)
        cnt = cnt + plsc.all_reduce_population_count(fresh)
        return cnt

    def compute(b, k):
        len_vec = seq_len_splat(b)
        fzero = jnp.zeros((L,), jnp.float32)
        acc0 = (fzero,) * ND

        def add_body(s, acc):
            m = jnp.full((L,), s, jnp.int32) < len_vec
            return tuple(acc[d] + jnp.where(m, rows_v[k, s, pl.ds(d * L, L)],
                                            fzero)
                         for d in range(ND))

        acc = lax.fori_loop(0, S, add_body, acc0)
        inv_v = 1.0 / len_vec.astype(jnp.float32)
        for d in range(ND):
            out_v[b, pl.ds(d * L, L)] = acc[d] * inv_v

    start_gather(0, 0)

    def outer(i, carry):
        for j in range(2):
            b = i * 2 + j

            @pl.when(b + 1 < BPW)
            def _():
                start_gather(b + 1, (j + 1) % 2)

            wait_gather(b, j)
            compute(b, j)
        return carry

    lax.fori_loop(0, BPW // 2, outer, 0)
    pltpu.sync_copy(out_v, out_hbm.at[pl.ds(base, BPW), :])


def kernel(input_ids, table):
    ids = input_ids.astype(jnp.int32)
    table = table.astype(jnp.float32)
    mesh = plsc.VectorSubcoreMesh(core_axis_name="c", subcore_axis_name="s")
    f = pl.kernel(
        _encoder_body,
        out_type=jax.ShapeDtypeStruct((B, D), jnp.float32),
        mesh=mesh,
        compiler_params=pltpu.CompilerParams(needs_layout_passes=False),
        scratch_types=[
            pltpu.VMEM((BPW, S), jnp.int32),
            pltpu.VMEM((2, S, D), jnp.float32),
            pltpu.VMEM((BPW, D), jnp.float32),
            pltpu.SemaphoreType.DMA,
            pltpu.SemaphoreType.DMA,
        ],
    )
    return f(ids, table)
